# Initial kernel scaffold; baseline (speedup 1.0000x reference)
import jax
import jax.numpy as jnp
from jax.experimental import pallas as pl


def kernel(flat, cu_seqlens, W1, b1, W2, b2, cluster_weights, cluster_weights2, hidden1_weights, gamma1, beta1, gamma2, beta2, gating_weights, gamma_g, beta_g):
    raise NotImplementedError("write your pallas kernel here")


# speedup vs baseline: 2.5621x; 2.5621x over previous
import functools

import jax
import jax.numpy as jnp
from jax.experimental import pallas as pl
from jax.experimental.pallas import tpu as pltpu

_TILE = 512
_M_PAD = 4096


def _fused(cu_ref, flat_ref, W1_ref, b1_ref, W2_ref, b2_ref, cw_ref, cw2_ref,
           h3_ref, g1_ref, be1_ref, g2_ref, be2_ref, gw_ref, gg_ref, bg_ref,
           out_ref,
           feats_s, act_s, sum_s, sumsq_s, stats_s, vlad_s, asum_s, cnt_s,
           *, n_tiles, n_batch, bm_rows):
    i = pl.program_id(0)

    @pl.when(i < n_tiles)
    def _phase_a():
        x = flat_ref[...]
        h = jnp.maximum(
            jnp.dot(x, W1_ref[...], preferred_element_type=jnp.float32)
            + b1_ref[...], 0.0)
        f = jnp.maximum(
            jnp.dot(h, W2_ref[...], preferred_element_type=jnp.float32)
            + b2_ref[...], 0.0)
        a = jnp.dot(f, cw_ref[...], preferred_element_type=jnp.float32)
        feats_s[pl.ds(i * _TILE, _TILE), :] = f
        act_s[pl.ds(i * _TILE, _TILE), :] = a

        @pl.when(i == 0)
        def _():
            sum_s[...] = jnp.zeros_like(sum_s)
            sumsq_s[...] = jnp.zeros_like(sumsq_s)

        sum_s[...] = sum_s[...] + jnp.sum(a, axis=0, keepdims=True)
        sumsq_s[...] = sumsq_s[...] + jnp.sum(a * a, axis=0, keepdims=True)

        @pl.when(i == n_tiles - 1)
        def _():
            mean = sum_s[...] / bm_rows
            var = sumsq_s[...] / bm_rows - mean * mean
            stats_s[0:1, :] = mean
            stats_s[1:2, :] = jax.lax.rsqrt(var + 1e-5)

    @pl.when(jnp.logical_and(i >= n_tiles, i < 2 * n_tiles))
    def _phase_b():
        t = i - n_tiles
        f = feats_s[pl.ds(t * _TILE, _TILE), :]
        a = act_s[pl.ds(t * _TILE, _TILE), :]
        mean = stats_s[0:1, :]
        inv = stats_s[1:2, :]
        an = (a - mean) * inv * g1_ref[...] + be1_ref[...]
        an = an - jnp.max(an, axis=-1, keepdims=True)
        e = jnp.exp(an)
        s = e / jnp.sum(e, axis=-1, keepdims=True)
        pv = jax.lax.dot_general(f, s, (((0,), (0,)), ((), ())),
                                 preferred_element_type=jnp.float32)
        ps = jnp.sum(s, axis=0, keepdims=True)

        tstart = t * _TILE
        seg = jnp.int32(0)
        for j in range(1, n_batch):
            seg = seg + jnp.where(tstart >= cu_ref[j], 1, 0).astype(jnp.int32)
        onehot = (jax.lax.broadcasted_iota(jnp.int32, (n_batch, 1), 0)
                  == seg).astype(jnp.float32)

        @pl.when(t == 0)
        def _():
            vlad_s[...] = jnp.zeros_like(vlad_s)
            asum_s[...] = jnp.zeros_like(asum_s)
            cnt_s[...] = jnp.zeros_like(cnt_s)

        vlad_s[...] = vlad_s[...] + onehot[:, :, None] * pv[None, :, :]
        asum_s[...] = asum_s[...] + onehot * ps
        cnt_s[...] = cnt_s[...] + onehot

    @pl.when(i == 2 * n_tiles)
    def _phase_c():
        mean = stats_s[0:1, :]
        inv = stats_s[1:2, :]
        a0 = (0.0 - mean) * inv * g1_ref[...] + be1_ref[...]
        a0 = a0 - jnp.max(a0, axis=-1, keepdims=True)
        e0 = jnp.exp(a0)
        p = e0 / jnp.sum(e0, axis=-1, keepdims=True)
        npad = jnp.float32(_M_PAD) - jnp.float32(_TILE) * cnt_s[...]
        a_sum = asum_s[...] + npad * p
        intra = vlad_s[...] - a_sum[:, None, :] * cw2_ref[...][None, :, :]
        n1 = jnp.sqrt(jnp.sum(intra * intra, axis=1, keepdims=True))
        y = intra / jnp.maximum(n1, 1e-12)
        n2 = jnp.sqrt(jnp.sum(y * y, axis=(1, 2), keepdims=True))
        y = y / jnp.maximum(n2, 1e-12)
        z = jax.lax.dot_general(y, h3_ref[...], (((2,), (1,)), ((1,), (0,))),
                                preferred_element_type=jnp.float32)
        out0 = jnp.sum(z, axis=0)
        mu = jnp.mean(out0, axis=0, keepdims=True)
        v = jnp.mean((out0 - mu) * (out0 - mu), axis=0, keepdims=True)
        o = (out0 - mu) * jax.lax.rsqrt(v + 1e-5) * g2_ref[...] + be2_ref[...]
        gt = jnp.dot(o, gw_ref[...], preferred_element_type=jnp.float32)
        mug = jnp.mean(gt, axis=0, keepdims=True)
        vg = jnp.mean((gt - mug) * (gt - mug), axis=0, keepdims=True)
        gn = (gt - mug) * jax.lax.rsqrt(vg + 1e-5) * gg_ref[...] + bg_ref[...]
        out_ref[...] = o * (1.0 / (1.0 + jnp.exp(-gn)))


@jax.jit
def kernel(flat, cu_seqlens, W1, b1, W2, b2, cluster_weights,
           cluster_weights2, hidden1_weights, gamma1, beta1, gamma2, beta2,
           gating_weights, gamma_g, beta_g):
    total, in_dim = flat.shape
    f_dim = W1.shape[1]
    c_dim = cluster_weights.shape[1]
    out_dim = hidden1_weights.shape[1]
    n_batch = cu_seqlens.shape[0] - 1
    n_tiles = total // _TILE
    grid = (2 * n_tiles + 1,)

    h3 = hidden1_weights.reshape(f_dim, c_dim, out_dim)
    cw2 = cluster_weights2.reshape(f_dim, c_dim)

    full = lambda *shape: pl.BlockSpec(shape, lambda i: (0,) * len(shape))
    body = functools.partial(_fused, n_tiles=n_tiles, n_batch=n_batch,
                             bm_rows=float(n_batch * _M_PAD))
    return pl.pallas_call(
        body,
        grid=grid,
        in_specs=[
            pl.BlockSpec(memory_space=pltpu.SMEM),
            pl.BlockSpec((_TILE, in_dim),
                         lambda i: (jnp.minimum(i, n_tiles - 1), 0)),
            full(in_dim, f_dim),
            full(1, f_dim),
            full(f_dim, f_dim),
            full(1, f_dim),
            full(f_dim, c_dim),
            full(f_dim, c_dim),
            full(f_dim, c_dim, out_dim),
            full(1, c_dim),
            full(1, c_dim),
            full(1, out_dim),
            full(1, out_dim),
            full(out_dim, out_dim),
            full(1, out_dim),
            full(1, out_dim),
        ],
        out_specs=full(n_batch, out_dim),
        out_shape=jax.ShapeDtypeStruct((n_batch, out_dim), jnp.float32),
        scratch_shapes=[
            pltpu.VMEM((total, f_dim), jnp.float32),
            pltpu.VMEM((total, c_dim), jnp.float32),
            pltpu.VMEM((1, c_dim), jnp.float32),
            pltpu.VMEM((1, c_dim), jnp.float32),
            pltpu.VMEM((2, c_dim), jnp.float32),
            pltpu.VMEM((n_batch, f_dim, c_dim), jnp.float32),
            pltpu.VMEM((n_batch, c_dim), jnp.float32),
            pltpu.VMEM((n_batch, 1), jnp.float32),
        ],
        compiler_params=pltpu.CompilerParams(
            dimension_semantics=("arbitrary",)),
    )(cu_seqlens, flat, W1, b1.reshape(1, f_dim), W2, b2.reshape(1, f_dim),
      cluster_weights, cw2, h3, gamma1.reshape(1, c_dim),
      beta1.reshape(1, c_dim), gamma2.reshape(1, out_dim),
      beta2.reshape(1, out_dim), gating_weights, gamma_g.reshape(1, out_dim),
      beta_g.reshape(1, out_dim))


# speedup vs baseline: 3.2934x; 1.2854x over previous
import functools

import jax
import jax.numpy as jnp
from jax.experimental import pallas as pl
from jax.experimental.pallas import tpu as pltpu

_TILE = 512
_ABLK = 4096
_M_PAD = 4096


def _fused(cu_ref, flat_ref, W1_ref, b1_ref, W2_ref, b2_ref, cw_ref, cw2_ref,
           h3_ref, g1_ref, be1_ref, g2_ref, be2_ref, gw_ref, gg_ref, bg_ref,
           out_ref,
           feats_s, act_s, sum_s, sumsq_s,
           *, n_a, n_tiles, n_batch, bm_rows):
    i = pl.program_id(0)
    f_dim = feats_s.shape[1]
    c_dim = act_s.shape[1]
    sub = _ABLK // _TILE

    @pl.when(i < n_a)
    def _phase_a():
        W1 = W1_ref[...]
        b1 = b1_ref[...]
        W2 = W2_ref[...]
        b2 = b2_ref[...]
        cw = cw_ref[...]

        def body(k, carry):
            ssum, ssq = carry
            x = flat_ref[pl.ds(k * _TILE, _TILE), :]
            h = jnp.maximum(
                jnp.dot(x, W1, preferred_element_type=jnp.float32) + b1, 0.0)
            f = jnp.maximum(
                jnp.dot(h, W2, preferred_element_type=jnp.float32) + b2, 0.0)
            a = jnp.dot(f, cw, preferred_element_type=jnp.float32)
            base = i * _ABLK + k * _TILE
            feats_s[pl.ds(base, _TILE), :] = f
            act_s[pl.ds(base, _TILE), :] = a
            return (ssum + jnp.sum(a, axis=0, keepdims=True),
                    ssq + jnp.sum(a * a, axis=0, keepdims=True))

        z = jnp.zeros((1, c_dim), jnp.float32)
        ssum, ssq = jax.lax.fori_loop(0, sub, body, (z, z))

        @pl.when(i == 0)
        def _():
            sum_s[...] = ssum
            sumsq_s[...] = ssq

        @pl.when(i > 0)
        def _():
            sum_s[...] = sum_s[...] + ssum
            sumsq_s[...] = sumsq_s[...] + ssq

    @pl.when(i == n_a)
    def _phase_z():
        mean = sum_s[...] / bm_rows
        var = sumsq_s[...] / bm_rows - mean * mean
        inv = jax.lax.rsqrt(var + 1e-5)
        g1 = g1_ref[...]
        be1 = be1_ref[...]
        kscale = inv * g1
        kbias = be1 - mean * kscale

        def body(t, carry):
            vlad, asum, cnt = carry
            f = feats_s[pl.ds(t * _TILE, _TILE), :]
            a = act_s[pl.ds(t * _TILE, _TILE), :]
            an = a * kscale + kbias
            an = an - jnp.max(an, axis=-1, keepdims=True)
            e = jnp.exp(an)
            s = e / jnp.sum(e, axis=-1, keepdims=True)
            pv = jax.lax.dot_general(f, s, (((0,), (0,)), ((), ())),
                                     preferred_element_type=jnp.float32)
            ps = jnp.sum(s, axis=0, keepdims=True)
            tstart = t * _TILE
            seg = jnp.int32(0)
            for j in range(1, n_batch):
                seg = seg + jnp.where(tstart >= cu_ref[j], 1, 0).astype(
                    jnp.int32)
            onehot = (jax.lax.broadcasted_iota(jnp.int32, (n_batch, 1), 0)
                      == seg).astype(jnp.float32)
            return (vlad + onehot[:, :, None] * pv[None, :, :],
                    asum + onehot * ps, cnt + onehot)

        vlad, asum, cnt = jax.lax.fori_loop(
            0, n_tiles, body,
            (jnp.zeros((n_batch, f_dim, c_dim), jnp.float32),
             jnp.zeros((n_batch, c_dim), jnp.float32),
             jnp.zeros((n_batch, 1), jnp.float32)))

        a0 = 0.0 * kscale + kbias
        a0 = a0 - jnp.max(a0, axis=-1, keepdims=True)
        e0 = jnp.exp(a0)
        p = e0 / jnp.sum(e0, axis=-1, keepdims=True)
        npad = jnp.float32(_M_PAD) - jnp.float32(_TILE) * cnt
        a_sum = asum + npad * p
        intra = vlad - a_sum[:, None, :] * cw2_ref[...][None, :, :]
        n1 = jnp.sqrt(jnp.sum(intra * intra, axis=1, keepdims=True))
        y = intra / jnp.maximum(n1, 1e-12)
        n2 = jnp.sqrt(jnp.sum(y * y, axis=(1, 2), keepdims=True))
        y = y / jnp.maximum(n2, 1e-12)
        z = jax.lax.dot_general(y, h3_ref[...], (((2,), (1,)), ((1,), (0,))),
                                preferred_element_type=jnp.float32)
        out0 = jnp.sum(z, axis=0)
        mu = jnp.mean(out0, axis=0, keepdims=True)
        v = jnp.mean((out0 - mu) * (out0 - mu), axis=0, keepdims=True)
        o = (out0 - mu) * jax.lax.rsqrt(v + 1e-5) * g2_ref[...] + be2_ref[...]
        gt = jnp.dot(o, gw_ref[...], preferred_element_type=jnp.float32)
        mug = jnp.mean(gt, axis=0, keepdims=True)
        vg = jnp.mean((gt - mug) * (gt - mug), axis=0, keepdims=True)
        gn = (gt - mug) * jax.lax.rsqrt(vg + 1e-5) * gg_ref[...] + bg_ref[...]
        out_ref[...] = o * (1.0 / (1.0 + jnp.exp(-gn)))


@jax.jit
def kernel(flat, cu_seqlens, W1, b1, W2, b2, cluster_weights,
           cluster_weights2, hidden1_weights, gamma1, beta1, gamma2, beta2,
           gating_weights, gamma_g, beta_g):
    total, in_dim = flat.shape
    f_dim = W1.shape[1]
    c_dim = cluster_weights.shape[1]
    out_dim = hidden1_weights.shape[1]
    n_batch = cu_seqlens.shape[0] - 1
    n_tiles = total // _TILE
    n_a = total // _ABLK
    grid = (n_a + 1,)

    h3 = hidden1_weights.reshape(f_dim, c_dim, out_dim)
    cw2 = cluster_weights2.reshape(f_dim, c_dim)

    full = lambda *shape: pl.BlockSpec(shape, lambda i: (0,) * len(shape))
    body = functools.partial(_fused, n_a=n_a, n_tiles=n_tiles,
                             n_batch=n_batch,
                             bm_rows=float(n_batch * _M_PAD))
    return pl.pallas_call(
        body,
        grid=grid,
        in_specs=[
            pl.BlockSpec(memory_space=pltpu.SMEM),
            pl.BlockSpec((_ABLK, in_dim),
                         lambda i: (jnp.minimum(i, n_a - 1), 0)),
            full(in_dim, f_dim),
            full(1, f_dim),
            full(f_dim, f_dim),
            full(1, f_dim),
            full(f_dim, c_dim),
            full(f_dim, c_dim),
            full(f_dim, c_dim, out_dim),
            full(1, c_dim),
            full(1, c_dim),
            full(1, out_dim),
            full(1, out_dim),
            full(out_dim, out_dim),
            full(1, out_dim),
            full(1, out_dim),
        ],
        out_specs=full(n_batch, out_dim),
        out_shape=jax.ShapeDtypeStruct((n_batch, out_dim), jnp.float32),
        scratch_shapes=[
            pltpu.VMEM((total, f_dim), jnp.float32),
            pltpu.VMEM((total, c_dim), jnp.float32),
            pltpu.VMEM((1, c_dim), jnp.float32),
            pltpu.VMEM((1, c_dim), jnp.float32),
        ],
        compiler_params=pltpu.CompilerParams(
            dimension_semantics=("arbitrary",)),
    )(cu_seqlens, flat, W1, b1.reshape(1, f_dim), W2, b2.reshape(1, f_dim),
      cluster_weights, cw2, h3, gamma1.reshape(1, c_dim),
      beta1.reshape(1, c_dim), gamma2.reshape(1, out_dim),
      beta2.reshape(1, out_dim), gating_weights, gamma_g.reshape(1, out_dim),
      beta_g.reshape(1, out_dim))


# speedup vs baseline: 4.1533x; 1.2611x over previous
import functools

import jax
import jax.numpy as jnp
from jax.experimental import pallas as pl
from jax.experimental.pallas import tpu as pltpu

_TILE = 512
_ABLK = 4096
_M_PAD = 4096


def _fused(cu_ref, flat_ref, W1_ref, b1_ref, W2_ref, b2_ref, cw_ref, cw2_ref,
           h3_ref, g1_ref, be1_ref, g2_ref, be2_ref, gw_ref, gg_ref, bg_ref,
           out_ref,
           feats_s, act_s, sum_s, sumsq_s,
           *, n_a, n_tiles, n_batch, bm_rows):
    i = pl.program_id(0)
    f_dim = feats_s.shape[1]
    c_dim = act_s.shape[1]
    sub = _ABLK // _TILE

    @pl.when(i < n_a)
    def _phase_a():
        W1 = W1_ref[...]
        b1 = b1_ref[...]
        W2 = W2_ref[...]
        b2 = b2_ref[...]
        cw = cw_ref[...]

        def one_tile(k):
            x = flat_ref[pl.ds(k * _TILE, _TILE), :]
            h = jnp.maximum(
                jnp.dot(x, W1, preferred_element_type=jnp.float32) + b1, 0.0)
            f = jnp.maximum(
                jnp.dot(h, W2, preferred_element_type=jnp.float32) + b2, 0.0)
            a = jnp.dot(f, cw, preferred_element_type=jnp.float32)
            base = i * _ABLK + k * _TILE
            feats_s[pl.ds(base, _TILE), :] = f
            act_s[pl.ds(base, _TILE), :] = a
            return (jnp.sum(a, axis=0, keepdims=True),
                    jnp.sum(a * a, axis=0, keepdims=True))

        def body(k2, carry):
            ssum, ssq = carry
            s0, q0 = one_tile(2 * k2)
            s1, q1 = one_tile(2 * k2 + 1)
            return (ssum + (s0 + s1), ssq + (q0 + q1))

        z = jnp.zeros((1, c_dim), jnp.float32)
        ssum, ssq = jax.lax.fori_loop(0, sub // 2, body, (z, z))

        @pl.when(i == 0)
        def _():
            sum_s[...] = ssum
            sumsq_s[...] = ssq

        @pl.when(i > 0)
        def _():
            sum_s[...] = sum_s[...] + ssum
            sumsq_s[...] = sumsq_s[...] + ssq

    @pl.when(i == n_a)
    def _phase_z():
        mean = sum_s[...] / bm_rows
        var = sumsq_s[...] / bm_rows - mean * mean
        inv = jax.lax.rsqrt(var + 1e-5)
        g1 = g1_ref[...]
        be1 = be1_ref[...]
        kscale = inv * g1
        kbias = be1 - mean * kscale

        def one_tile(t):
            f = feats_s[pl.ds(t * _TILE, _TILE), :]
            a = act_s[pl.ds(t * _TILE, _TILE), :]
            an = a * kscale + kbias
            an = an - jnp.max(an, axis=-1, keepdims=True)
            e = jnp.exp(an)
            s = e / jnp.sum(e, axis=-1, keepdims=True)
            pv = jax.lax.dot_general(f, s, (((0,), (0,)), ((), ())),
                                     preferred_element_type=jnp.float32)
            ps = jnp.sum(s, axis=0, keepdims=True)
            tstart = t * _TILE
            seg = jnp.int32(0)
            for j in range(1, n_batch):
                seg = seg + jnp.where(tstart >= cu_ref[j], 1, 0).astype(
                    jnp.int32)
            onehot = (jax.lax.broadcasted_iota(jnp.int32, (n_batch, 1), 0)
                      == seg).astype(jnp.float32)
            return pv, ps, onehot

        def body(t2, carry):
            vlad, asum, cnt = carry
            pv0, ps0, oh0 = one_tile(2 * t2)
            pv1, ps1, oh1 = one_tile(2 * t2 + 1)
            vlad = vlad + (oh0[:, :, None] * pv0[None, :, :]
                           + oh1[:, :, None] * pv1[None, :, :])
            return (vlad, asum + (oh0 * ps0 + oh1 * ps1),
                    cnt + (oh0 + oh1))

        vlad, asum, cnt = jax.lax.fori_loop(
            0, n_tiles // 2, body,
            (jnp.zeros((n_batch, f_dim, c_dim), jnp.float32),
             jnp.zeros((n_batch, c_dim), jnp.float32),
             jnp.zeros((n_batch, 1), jnp.float32)))

        a0 = 0.0 * kscale + kbias
        a0 = a0 - jnp.max(a0, axis=-1, keepdims=True)
        e0 = jnp.exp(a0)
        p = e0 / jnp.sum(e0, axis=-1, keepdims=True)
        npad = jnp.float32(_M_PAD) - jnp.float32(_TILE) * cnt
        a_sum = asum + npad * p
        intra = vlad - a_sum[:, None, :] * cw2_ref[...][None, :, :]
        n1 = jnp.sqrt(jnp.sum(intra * intra, axis=1, keepdims=True))
        y = intra / jnp.maximum(n1, 1e-12)
        n2 = jnp.sqrt(jnp.sum(y * y, axis=(1, 2), keepdims=True))
        y = y / jnp.maximum(n2, 1e-12)
        z = jax.lax.dot_general(y, h3_ref[...], (((2,), (1,)), ((1,), (0,))),
                                preferred_element_type=jnp.float32)
        out0 = jnp.sum(z, axis=0)
        mu = jnp.mean(out0, axis=0, keepdims=True)
        v = jnp.mean((out0 - mu) * (out0 - mu), axis=0, keepdims=True)
        o = (out0 - mu) * jax.lax.rsqrt(v + 1e-5) * g2_ref[...] + be2_ref[...]
        gt = jnp.dot(o, gw_ref[...], preferred_element_type=jnp.float32)
        mug = jnp.mean(gt, axis=0, keepdims=True)
        vg = jnp.mean((gt - mug) * (gt - mug), axis=0, keepdims=True)
        gn = (gt - mug) * jax.lax.rsqrt(vg + 1e-5) * gg_ref[...] + bg_ref[...]
        out_ref[...] = o * (1.0 / (1.0 + jnp.exp(-gn)))


@jax.jit
def kernel(flat, cu_seqlens, W1, b1, W2, b2, cluster_weights,
           cluster_weights2, hidden1_weights, gamma1, beta1, gamma2, beta2,
           gating_weights, gamma_g, beta_g):
    total, in_dim = flat.shape
    f_dim = W1.shape[1]
    c_dim = cluster_weights.shape[1]
    out_dim = hidden1_weights.shape[1]
    n_batch = cu_seqlens.shape[0] - 1
    n_tiles = total // _TILE
    n_a = total // _ABLK
    grid = (n_a + 1,)

    h3 = hidden1_weights.reshape(f_dim, c_dim, out_dim)
    cw2 = cluster_weights2.reshape(f_dim, c_dim)

    full = lambda *shape: pl.BlockSpec(shape, lambda i: (0,) * len(shape))
    body = functools.partial(_fused, n_a=n_a, n_tiles=n_tiles,
                             n_batch=n_batch,
                             bm_rows=float(n_batch * _M_PAD))
    return pl.pallas_call(
        body,
        grid=grid,
        in_specs=[
            pl.BlockSpec(memory_space=pltpu.SMEM),
            pl.BlockSpec((_ABLK, in_dim),
                         lambda i: (jnp.minimum(i, n_a - 1), 0)),
            full(in_dim, f_dim),
            full(1, f_dim),
            full(f_dim, f_dim),
            full(1, f_dim),
            full(f_dim, c_dim),
            full(f_dim, c_dim),
            full(f_dim, c_dim, out_dim),
            full(1, c_dim),
            full(1, c_dim),
            full(1, out_dim),
            full(1, out_dim),
            full(out_dim, out_dim),
            full(1, out_dim),
            full(1, out_dim),
        ],
        out_specs=full(n_batch, out_dim),
        out_shape=jax.ShapeDtypeStruct((n_batch, out_dim), jnp.float32),
        scratch_shapes=[
            pltpu.VMEM((total, f_dim), jnp.float32),
            pltpu.VMEM((total, c_dim), jnp.float32),
            pltpu.VMEM((1, c_dim), jnp.float32),
            pltpu.VMEM((1, c_dim), jnp.float32),
        ],
        compiler_params=pltpu.CompilerParams(
            dimension_semantics=("arbitrary",)),
    )(cu_seqlens, flat, W1, b1.reshape(1, f_dim), W2, b2.reshape(1, f_dim),
      cluster_weights, cw2, h3, gamma1.reshape(1, c_dim),
      beta1.reshape(1, c_dim), gamma2.reshape(1, out_dim),
      beta2.reshape(1, out_dim), gating_weights, gamma_g.reshape(1, out_dim),
      beta_g.reshape(1, out_dim))


# speedup vs baseline: 4.8592x; 1.1700x over previous
import functools

import jax
import jax.numpy as jnp
from jax.experimental import pallas as pl
from jax.experimental.pallas import tpu as pltpu

_TILE = 512
_ABLK = 4096
_M_PAD = 4096


def _fused(cu_ref, flat_ref, W1_ref, b1_ref, W2_ref, b2_ref, cw_ref, cw2_ref,
           h3_ref, g1_ref, be1_ref, g2_ref, be2_ref, gw_ref, gg_ref, bg_ref,
           out_ref,
           feats_s, act_s, sum_s, sumsq_s, h3_s, h3_sem,
           *, n_a, n_tiles, n_batch, bm_rows):
    i = pl.program_id(0)
    f_dim = feats_s.shape[1]
    c_dim = act_s.shape[1]
    sub = _ABLK // _TILE

    h3_copy = pltpu.make_async_copy(h3_ref, h3_s, h3_sem)

    @pl.when(i == 0)
    def _start_h3():
        h3_copy.start()

    @pl.when(i < n_a)
    def _phase_a():
        W1 = W1_ref[...]
        b1 = b1_ref[...]
        W2 = W2_ref[...]
        b2 = b2_ref[...]
        cw = cw_ref[...]

        def one_tile(k):
            x = flat_ref[pl.ds(k * _TILE, _TILE), :]
            h = jnp.maximum(
                jnp.dot(x, W1, preferred_element_type=jnp.float32) + b1, 0.0)
            f = jnp.maximum(
                jnp.dot(h, W2, preferred_element_type=jnp.float32) + b2, 0.0)
            a = jnp.dot(f, cw, preferred_element_type=jnp.float32)
            base = i * _ABLK + k * _TILE
            feats_s[pl.ds(base, _TILE), :] = f
            act_s[pl.ds(base, _TILE), :] = a
            return (jnp.sum(a, axis=0, keepdims=True),
                    jnp.sum(a * a, axis=0, keepdims=True))

        def body(k4, carry):
            ssum, ssq = carry
            s0, q0 = one_tile(4 * k4)
            s1, q1 = one_tile(4 * k4 + 1)
            s2, q2 = one_tile(4 * k4 + 2)
            s3, q3 = one_tile(4 * k4 + 3)
            return (ssum + ((s0 + s1) + (s2 + s3)),
                    ssq + ((q0 + q1) + (q2 + q3)))

        z = jnp.zeros((1, c_dim), jnp.float32)
        ssum, ssq = jax.lax.fori_loop(0, sub // 4, body, (z, z))

        @pl.when(i == 0)
        def _():
            sum_s[...] = ssum
            sumsq_s[...] = ssq

        @pl.when(i > 0)
        def _():
            sum_s[...] = sum_s[...] + ssum
            sumsq_s[...] = sumsq_s[...] + ssq

    @pl.when(i == n_a)
    def _phase_z():
        h3_copy.wait()
        mean = sum_s[...] / bm_rows
        var = sumsq_s[...] / bm_rows - mean * mean
        inv = jax.lax.rsqrt(var + 1e-5)
        g1 = g1_ref[...]
        be1 = be1_ref[...]
        kscale = inv * g1
        kbias = be1 - mean * kscale

        def one_tile(t):
            f = feats_s[pl.ds(t * _TILE, _TILE), :]
            a = act_s[pl.ds(t * _TILE, _TILE), :]
            an = a * kscale + kbias
            an = an - jnp.max(an, axis=-1, keepdims=True)
            e = jnp.exp(an)
            s = e / jnp.sum(e, axis=-1, keepdims=True)
            pv = jax.lax.dot_general(f, s, (((0,), (0,)), ((), ())),
                                     preferred_element_type=jnp.float32)
            ps = jnp.sum(s, axis=0, keepdims=True)
            tstart = t * _TILE
            seg = jnp.int32(0)
            for j in range(1, n_batch):
                seg = seg + jnp.where(tstart >= cu_ref[j], 1, 0).astype(
                    jnp.int32)
            onehot = (jax.lax.broadcasted_iota(jnp.int32, (n_batch, 1), 0)
                      == seg).astype(jnp.float32)
            return pv, ps, onehot

        def body(t4, carry):
            vlad, asum, cnt = carry
            pv0, ps0, oh0 = one_tile(4 * t4)
            pv1, ps1, oh1 = one_tile(4 * t4 + 1)
            pv2, ps2, oh2 = one_tile(4 * t4 + 2)
            pv3, ps3, oh3 = one_tile(4 * t4 + 3)
            vlad = vlad + ((oh0[:, :, None] * pv0[None, :, :]
                            + oh1[:, :, None] * pv1[None, :, :])
                           + (oh2[:, :, None] * pv2[None, :, :]
                              + oh3[:, :, None] * pv3[None, :, :]))
            return (vlad, asum + ((oh0 * ps0 + oh1 * ps1)
                                  + (oh2 * ps2 + oh3 * ps3)),
                    cnt + ((oh0 + oh1) + (oh2 + oh3)))

        vlad, asum, cnt = jax.lax.fori_loop(
            0, n_tiles // 4, body,
            (jnp.zeros((n_batch, f_dim, c_dim), jnp.float32),
             jnp.zeros((n_batch, c_dim), jnp.float32),
             jnp.zeros((n_batch, 1), jnp.float32)))

        a0 = 0.0 * kscale + kbias
        a0 = a0 - jnp.max(a0, axis=-1, keepdims=True)
        e0 = jnp.exp(a0)
        p = e0 / jnp.sum(e0, axis=-1, keepdims=True)
        npad = jnp.float32(_M_PAD) - jnp.float32(_TILE) * cnt
        a_sum = asum + npad * p
        intra = vlad - a_sum[:, None, :] * cw2_ref[...][None, :, :]
        n1 = jnp.sqrt(jnp.sum(intra * intra, axis=1, keepdims=True))
        y = intra / jnp.maximum(n1, 1e-12)
        n2 = jnp.sqrt(jnp.sum(y * y, axis=(1, 2), keepdims=True))
        y = y / jnp.maximum(n2, 1e-12)
        z = jax.lax.dot_general(y, h3_s[...], (((2,), (1,)), ((1,), (0,))),
                                preferred_element_type=jnp.float32)
        out0 = jnp.sum(z, axis=0)
        mu = jnp.mean(out0, axis=0, keepdims=True)
        v = jnp.mean((out0 - mu) * (out0 - mu), axis=0, keepdims=True)
        o = (out0 - mu) * jax.lax.rsqrt(v + 1e-5) * g2_ref[...] + be2_ref[...]
        gt = jnp.dot(o, gw_ref[...], preferred_element_type=jnp.float32)
        mug = jnp.mean(gt, axis=0, keepdims=True)
        vg = jnp.mean((gt - mug) * (gt - mug), axis=0, keepdims=True)
        gn = (gt - mug) * jax.lax.rsqrt(vg + 1e-5) * gg_ref[...] + bg_ref[...]
        out_ref[...] = o * (1.0 / (1.0 + jnp.exp(-gn)))


@jax.jit
def kernel(flat, cu_seqlens, W1, b1, W2, b2, cluster_weights,
           cluster_weights2, hidden1_weights, gamma1, beta1, gamma2, beta2,
           gating_weights, gamma_g, beta_g):
    total, in_dim = flat.shape
    f_dim = W1.shape[1]
    c_dim = cluster_weights.shape[1]
    out_dim = hidden1_weights.shape[1]
    n_batch = cu_seqlens.shape[0] - 1
    n_tiles = total // _TILE
    n_a = total // _ABLK
    grid = (n_a + 1,)

    h3 = hidden1_weights.reshape(f_dim, c_dim, out_dim)
    cw2 = cluster_weights2.reshape(f_dim, c_dim)

    full = lambda *shape: pl.BlockSpec(shape, lambda i: (0,) * len(shape))
    body = functools.partial(_fused, n_a=n_a, n_tiles=n_tiles,
                             n_batch=n_batch,
                             bm_rows=float(n_batch * _M_PAD))
    return pl.pallas_call(
        body,
        grid=grid,
        in_specs=[
            pl.BlockSpec(memory_space=pltpu.SMEM),
            pl.BlockSpec((_ABLK, in_dim),
                         lambda i: (jnp.minimum(i, n_a - 1), 0)),
            full(in_dim, f_dim),
            full(1, f_dim),
            full(f_dim, f_dim),
            full(1, f_dim),
            full(f_dim, c_dim),
            full(f_dim, c_dim),
            pl.BlockSpec(memory_space=pltpu.MemorySpace.HBM),
            full(1, c_dim),
            full(1, c_dim),
            full(1, out_dim),
            full(1, out_dim),
            full(out_dim, out_dim),
            full(1, out_dim),
            full(1, out_dim),
        ],
        out_specs=full(n_batch, out_dim),
        out_shape=jax.ShapeDtypeStruct((n_batch, out_dim), jnp.float32),
        scratch_shapes=[
            pltpu.VMEM((total, f_dim), jnp.float32),
            pltpu.VMEM((total, c_dim), jnp.float32),
            pltpu.VMEM((1, c_dim), jnp.float32),
            pltpu.VMEM((1, c_dim), jnp.float32),
            pltpu.VMEM((f_dim, c_dim, out_dim), jnp.float32),
            pltpu.SemaphoreType.DMA,
        ],
        compiler_params=pltpu.CompilerParams(
            dimension_semantics=("arbitrary",)),
    )(cu_seqlens, flat, W1, b1.reshape(1, f_dim), W2, b2.reshape(1, f_dim),
      cluster_weights, cw2, h3, gamma1.reshape(1, c_dim),
      beta1.reshape(1, c_dim), gamma2.reshape(1, out_dim),
      beta2.reshape(1, out_dim), gating_weights, gamma_g.reshape(1, out_dim),
      beta_g.reshape(1, out_dim))
